# Initial kernel scaffold; baseline (speedup 1.0000x reference)
import jax
import jax.numpy as jnp
from jax.experimental import pallas as pl


def kernel(x, edge_index, batch, W_emb, b_emb, affinity_scores, Wv1, bv1, Wv2, bv2, Wm1, bm1, Wm2, bm2):
    raise NotImplementedError("write your pallas kernel here")


# speedup vs baseline: 3.0140x; 3.0140x over previous
import jax
import jax.numpy as jnp
from jax.experimental import pallas as pl
from jax.experimental.pallas import tpu as pltpu

HIDDEN = 512
NVN = 64
TOPK = 5
NG = 8
NP = 10240
BLK = 1024
ROWS = NG * NVN


def _emb_kernel(x_ref, we_ref, be_ref, amat_ref, h_ref, aff_ref):
    h = jnp.dot(x_ref[...], we_ref[...],
                preferred_element_type=jnp.float32) + be_ref[...]
    h_ref[...] = h
    aff_ref[...] = jnp.dot(h, amat_ref[...],
                           preferred_element_type=jnp.float32)


CHUNK = 512
NCH = NP // CHUNK


def _topk_kernel(aff_ref, batch_ref, w2_ref, denom_ref):
    neg_inf = jnp.float32(-jnp.inf)

    def chunk_at(c):
        a = aff_ref[pl.ds(c * CHUNK, CHUNK), :]
        b = batch_ref[pl.ds(c * CHUNK, CHUNK), :]
        gidx = (jax.lax.broadcasted_iota(jnp.int32, (CHUNK, NVN), 0)
                + c * CHUNK)
        return a, b, gidx

    vals = []
    idxs = []
    cnt = None
    for r in range(TOPK):

        def body(c, carry, _prev=tuple(idxs), _first=(r == 0)):
            m_run, i_run, cnt_run = carry
            a, b, gidx = chunk_at(c)
            nm, ni, nc = [], [], []
            for g in range(NG):
                mask = b == g
                am = jnp.where(mask, a, neg_inf)
                for p in _prev:
                    am = jnp.where(gidx == p[g:g + 1, :], neg_inf, am)
                cm = jnp.max(am, axis=0, keepdims=True)
                ci = jnp.min(
                    jnp.where((am == cm) & (cm > neg_inf), gidx, NP),
                    axis=0, keepdims=True)
                mr = m_run[g:g + 1, :]
                ir = i_run[g:g + 1, :]
                better = cm > mr
                tie = cm == mr
                nm.append(jnp.maximum(cm, mr))
                ni.append(jnp.where(better, ci,
                                    jnp.where(tie, jnp.minimum(ci, ir), ir)))
                if _first:
                    s = jnp.sum(mask.astype(jnp.float32))
                    nc.append(cnt_run[g:g + 1, :] + s)
                else:
                    nc.append(cnt_run[g:g + 1, :])
            return (jnp.concatenate(nm, 0), jnp.concatenate(ni, 0),
                    jnp.concatenate(nc, 0))

        init = (jnp.full((NG, NVN), neg_inf, jnp.float32),
                jnp.full((NG, NVN), NP, jnp.int32),
                cnt if cnt is not None else jnp.zeros((NG, NVN), jnp.float32))
        m_fin, i_fin, cnt = jax.lax.fori_loop(0, NCH, body, init)
        vals.append(m_fin)
        idxs.append(i_fin)

    M = jnp.maximum(vals[0], 0.0)
    base = jnp.exp(-M)
    wvals = []
    denom = cnt * base
    for r in range(TOPK):
        valid = vals[r] > neg_inf
        wv = jnp.where(valid, jnp.exp(vals[r] - M), 0.0)
        wvals.append(wv)
        denom = denom + jnp.where(valid, wv - base, 0.0)
    denom_ref[...] = denom

    def build(c, carry):
        a, b, gidx = chunk_at(c)
        w2c = jnp.zeros((CHUNK, NVN), jnp.float32)
        for g in range(NG):
            w2g = jnp.broadcast_to(base[g:g + 1, :], (CHUNK, NVN))
            for r in range(TOPK):
                w2g = jnp.where(gidx == idxs[r][g:g + 1, :],
                                wvals[r][g:g + 1, :], w2g)
            w2c = jnp.where(b == g, w2g, w2c)
        w2_ref[pl.ds(c * CHUNK, CHUNK), :] = w2c
        return carry

    jax.lax.fori_loop(0, NCH, build, 0)


def _agg_kernel(w2_ref, h_ref, batch_ref, out_ref):
    i = pl.program_id(0)

    @pl.when(i == 0)
    def _():
        out_ref[...] = jnp.zeros_like(out_ref)

    b = batch_ref[...]
    g_lo = b[0, 0]
    g_hi = jnp.minimum(b[BLK - 1, 0], NG - 1)
    w2 = w2_ref[...]
    h = h_ref[...]

    def body(g, carry):
        wm = jnp.where(b == g, w2, 0.0)
        part = jax.lax.dot_general(
            wm, h, (((0,), (0,)), ((), ())),
            preferred_element_type=jnp.float32)
        out_ref[pl.ds(g, 1)] = out_ref[pl.ds(g, 1)] + part[None]
        return carry

    jax.lax.fori_loop(g_lo, g_hi + 1, body, 0)


def _head_kernel(num_ref, den_ref, wv1_ref, bv1_ref, wv2_ref, bv2_ref,
                 wm1_ref, bm1_ref, wm2_ref, bm2_ref, out_ref):
    num = num_ref[...]
    den = den_ref[...]
    vn3 = num * (1.0 / den)[:, :, None]
    vn = vn3.reshape(ROWS, HIDDEN)
    z = jnp.maximum(jnp.dot(vn, wv1_ref[...],
                            preferred_element_type=jnp.float32)
                    + bv1_ref[...], 0.0)
    z = jnp.dot(z, wv2_ref[...],
                preferred_element_type=jnp.float32) + bv2_ref[...]
    row = jax.lax.broadcasted_iota(jnp.int32, (NG, ROWS), 0)
    col = jax.lax.broadcasted_iota(jnp.int32, (NG, ROWS), 1)
    pool = jnp.where(col // NVN == row, jnp.float32(1.0 / NVN), 0.0)
    gf = jnp.dot(pool, z, preferred_element_type=jnp.float32)
    y = jnp.maximum(jnp.dot(gf, wm1_ref[...],
                            preferred_element_type=jnp.float32)
                    + bm1_ref[...], 0.0)
    out_ref[...] = jnp.dot(y, wm2_ref[...],
                           preferred_element_type=jnp.float32) + bm2_ref[...]


def kernel(x, edge_index, batch, W_emb, b_emb, affinity_scores,
           Wv1, bv1, Wv2, bv2, Wm1, bm1, Wm2, bm2):
    n = x.shape[0]
    in_dim = x.shape[1]
    xp = jnp.pad(x, ((0, NP - n), (0, 0)))
    bp = jnp.pad(batch.astype(jnp.int32), (0, NP - n),
                 constant_values=NG)[:, None]
    amat = affinity_scores[0]

    nblk = NP // BLK
    h, aff = pl.pallas_call(
        _emb_kernel,
        grid=(nblk,),
        in_specs=[
            pl.BlockSpec((BLK, in_dim), lambda i: (i, 0)),
            pl.BlockSpec((in_dim, HIDDEN), lambda i: (0, 0)),
            pl.BlockSpec((1, HIDDEN), lambda i: (0, 0)),
            pl.BlockSpec((HIDDEN, NVN), lambda i: (0, 0)),
        ],
        out_specs=[
            pl.BlockSpec((BLK, HIDDEN), lambda i: (i, 0)),
            pl.BlockSpec((BLK, NVN), lambda i: (i, 0)),
        ],
        out_shape=[
            jax.ShapeDtypeStruct((NP, HIDDEN), jnp.float32),
            jax.ShapeDtypeStruct((NP, NVN), jnp.float32),
        ],
    )(xp, W_emb, b_emb[None, :], amat)

    w2, denom = pl.pallas_call(
        _topk_kernel,
        in_specs=[
            pl.BlockSpec((NP, NVN), lambda: (0, 0)),
            pl.BlockSpec((NP, 1), lambda: (0, 0)),
        ],
        out_specs=[
            pl.BlockSpec((NP, NVN), lambda: (0, 0)),
            pl.BlockSpec((NG, NVN), lambda: (0, 0)),
        ],
        out_shape=[
            jax.ShapeDtypeStruct((NP, NVN), jnp.float32),
            jax.ShapeDtypeStruct((NG, NVN), jnp.float32),
        ],
    )(aff, bp)

    numer = pl.pallas_call(
        _agg_kernel,
        grid=(nblk,),
        in_specs=[
            pl.BlockSpec((BLK, NVN), lambda i: (i, 0)),
            pl.BlockSpec((BLK, HIDDEN), lambda i: (i, 0)),
            pl.BlockSpec((BLK, 1), lambda i: (i, 0)),
        ],
        out_specs=pl.BlockSpec((NG, NVN, HIDDEN), lambda i: (0, 0, 0)),
        out_shape=jax.ShapeDtypeStruct((NG, NVN, HIDDEN), jnp.float32),
    )(w2, h, bp)

    out = pl.pallas_call(
        _head_kernel,
        in_specs=[
            pl.BlockSpec((NG, NVN, HIDDEN), lambda: (0, 0, 0)),
            pl.BlockSpec((NG, NVN), lambda: (0, 0)),
            pl.BlockSpec((HIDDEN, HIDDEN), lambda: (0, 0)),
            pl.BlockSpec((1, HIDDEN), lambda: (0, 0)),
            pl.BlockSpec((HIDDEN, HIDDEN), lambda: (0, 0)),
            pl.BlockSpec((1, HIDDEN), lambda: (0, 0)),
            pl.BlockSpec((HIDDEN, HIDDEN), lambda: (0, 0)),
            pl.BlockSpec((1, HIDDEN), lambda: (0, 0)),
            pl.BlockSpec((HIDDEN, 128), lambda: (0, 0)),
            pl.BlockSpec((1, 128), lambda: (0, 0)),
        ],
        out_specs=pl.BlockSpec((NG, 128), lambda: (0, 0)),
        out_shape=jax.ShapeDtypeStruct((NG, 128), jnp.float32),
    )(numer, denom, Wv1, bv1[None, :], Wv2, bv2[None, :],
      Wm1, bm1[None, :], Wm2, bm2[None, :])
    return out


# speedup vs baseline: 5.3235x; 1.7662x over previous
import jax
import jax.numpy as jnp
from jax.experimental import pallas as pl
from jax.experimental.pallas import tpu as pltpu

HIDDEN = 512
NVN = 64
TOPK = 5
NG = 8
NP = 10240
BLK = 1024
ROWS = NG * NVN


def _emb_kernel(x_ref, we_ref, be_ref, amat_ref, h_ref, aff_ref):
    h = jnp.dot(x_ref[...], we_ref[...],
                preferred_element_type=jnp.float32) + be_ref[...]
    h_ref[...] = h
    aff_ref[...] = jnp.dot(h, amat_ref[...],
                           preferred_element_type=jnp.float32)


CHUNK = 512
NCH = NP // CHUNK


def _topk_kernel(aff_ref, batch_ref, sb_ref, w2_ref, denom_ref):
    neg_inf = jnp.float32(-jnp.inf)
    iota_c = jax.lax.broadcasted_iota(jnp.int32, (CHUNK, NVN), 0)

    vals = []
    idxs = []
    bounds = [(sb_ref[g], sb_ref[g + 1]) for g in range(NG)]
    for r in range(TOPK):
        per_m, per_i = [], []
        for g in range(NG):
            s, e = bounds[g]
            prev = [idxs[q][g:g + 1, :] for q in range(r)]

            def body(c, carry, _s=s, _e=e, _prev=tuple(prev)):
                m_run, i_run = carry
                a = aff_ref[pl.ds(c * CHUNK, CHUNK), :]
                gidx = iota_c + c * CHUNK
                am = jnp.where((gidx >= _s) & (gidx < _e), a, neg_inf)
                for p in _prev:
                    am = jnp.where(gidx == p, neg_inf, am)
                cm = jnp.max(am, axis=0, keepdims=True)
                ci = jnp.min(
                    jnp.where((am == cm) & (cm > neg_inf), gidx, NP),
                    axis=0, keepdims=True)
                better = cm > m_run
                tie = cm == m_run
                return (jnp.maximum(cm, m_run),
                        jnp.where(better, ci,
                                  jnp.where(tie, jnp.minimum(ci, i_run),
                                            i_run)))

            c0 = s // CHUNK
            ce = jnp.maximum((e + CHUNK - 1) // CHUNK, c0)
            init = (jnp.full((1, NVN), neg_inf, jnp.float32),
                    jnp.full((1, NVN), NP, jnp.int32))
            m_g, i_g = jax.lax.fori_loop(c0, ce, body, init)
            per_m.append(m_g)
            per_i.append(i_g)
        vals.append(jnp.concatenate(per_m, 0))
        idxs.append(jnp.concatenate(per_i, 0))

    M = jnp.maximum(vals[0], 0.0)
    base = jnp.exp(-M)
    cnt = jnp.concatenate(
        [jnp.full((1, NVN), 1.0, jnp.float32) * (e - s).astype(jnp.float32)
         for (s, e) in bounds], 0)
    denom = cnt * base
    idxs_f = []
    for r in range(TOPK):
        valid = vals[r] > neg_inf
        wv = jnp.where(valid, jnp.exp(vals[r] - M), 0.0)
        denom = denom + jnp.where(valid, wv - base, 0.0)
        idxs_f.append(idxs[r].astype(jnp.float32))
    denom_ref[...] = denom

    g8 = jax.lax.broadcasted_iota(jnp.int32, (CHUNK, NG), 1)

    def build(c, carry):
        a = aff_ref[pl.ds(c * CHUNK, CHUNK), :]
        b = batch_ref[pl.ds(c * CHUNK, CHUNK), :]
        gidx_f = (iota_c + c * CHUNK).astype(jnp.float32)
        onehot = (b == g8).astype(jnp.float32)
        msel = jnp.dot(onehot, M, preferred_element_type=jnp.float32,
                       precision=jax.lax.Precision.HIGHEST)
        hit = jnp.zeros((CHUNK, NVN), jnp.bool_)
        for r in range(TOPK):
            isel = jnp.dot(onehot, idxs_f[r],
                           preferred_element_type=jnp.float32,
                           precision=jax.lax.Precision.HIGHEST)
            hit = hit | (gidx_f == isel)
        w2c = jnp.exp(jnp.where(hit, a, 0.0) - msel)
        w2c = jnp.where(b < NG, w2c, 0.0)
        w2_ref[pl.ds(c * CHUNK, CHUNK), :] = w2c
        return carry

    jax.lax.fori_loop(0, NCH, build, 0)


def _agg_kernel(w2_ref, h_ref, batch_ref, out_ref):
    i = pl.program_id(0)

    @pl.when(i == 0)
    def _():
        out_ref[...] = jnp.zeros_like(out_ref)

    b = batch_ref[...]
    g_lo = b[0, 0]
    g_hi = jnp.minimum(b[BLK - 1, 0], NG - 1)
    w2 = w2_ref[...]
    h = h_ref[...]

    def body(g, carry):
        wm = jnp.where(b == g, w2, 0.0)
        part = jax.lax.dot_general(
            wm, h, (((0,), (0,)), ((), ())),
            preferred_element_type=jnp.float32)
        out_ref[pl.ds(g, 1)] = out_ref[pl.ds(g, 1)] + part[None]
        return carry

    jax.lax.fori_loop(g_lo, g_hi + 1, body, 0)


def _head_kernel(num_ref, den_ref, wv1_ref, bv1_ref, wv2_ref, bv2_ref,
                 wm1_ref, bm1_ref, wm2_ref, bm2_ref, out_ref):
    num = num_ref[...]
    den = den_ref[...]
    vn3 = num * (1.0 / den)[:, :, None]
    vn = vn3.reshape(ROWS, HIDDEN)
    z = jnp.maximum(jnp.dot(vn, wv1_ref[...],
                            preferred_element_type=jnp.float32)
                    + bv1_ref[...], 0.0)
    z = jnp.dot(z, wv2_ref[...],
                preferred_element_type=jnp.float32) + bv2_ref[...]
    row = jax.lax.broadcasted_iota(jnp.int32, (NG, ROWS), 0)
    col = jax.lax.broadcasted_iota(jnp.int32, (NG, ROWS), 1)
    pool = jnp.where(col // NVN == row, jnp.float32(1.0 / NVN), 0.0)
    gf = jnp.dot(pool, z, preferred_element_type=jnp.float32)
    y = jnp.maximum(jnp.dot(gf, wm1_ref[...],
                            preferred_element_type=jnp.float32)
                    + bm1_ref[...], 0.0)
    out_ref[...] = jnp.dot(y, wm2_ref[...],
                           preferred_element_type=jnp.float32) + bm2_ref[...]


def kernel(x, edge_index, batch, W_emb, b_emb, affinity_scores,
           Wv1, bv1, Wv2, bv2, Wm1, bm1, Wm2, bm2):
    n = x.shape[0]
    in_dim = x.shape[1]
    xp = jnp.pad(x, ((0, NP - n), (0, 0)))
    bp = jnp.pad(batch.astype(jnp.int32), (0, NP - n),
                 constant_values=NG)[:, None]
    amat = affinity_scores[0]

    nblk = NP // BLK
    h, aff = pl.pallas_call(
        _emb_kernel,
        grid=(nblk,),
        in_specs=[
            pl.BlockSpec((BLK, in_dim), lambda i: (i, 0)),
            pl.BlockSpec((in_dim, HIDDEN), lambda i: (0, 0)),
            pl.BlockSpec((1, HIDDEN), lambda i: (0, 0)),
            pl.BlockSpec((HIDDEN, NVN), lambda i: (0, 0)),
        ],
        out_specs=[
            pl.BlockSpec((BLK, HIDDEN), lambda i: (i, 0)),
            pl.BlockSpec((BLK, NVN), lambda i: (i, 0)),
        ],
        out_shape=[
            jax.ShapeDtypeStruct((NP, HIDDEN), jnp.float32),
            jax.ShapeDtypeStruct((NP, NVN), jnp.float32),
        ],
    )(xp, W_emb, b_emb[None, :], amat)

    sb = jnp.searchsorted(
        batch.astype(jnp.int32),
        jnp.arange(NG + 1, dtype=jnp.int32)).astype(jnp.int32)
    sb = jnp.pad(sb, (0, 16 - (NG + 1)))

    w2, denom = pl.pallas_call(
        _topk_kernel,
        in_specs=[
            pl.BlockSpec((NP, NVN), lambda: (0, 0)),
            pl.BlockSpec((NP, 1), lambda: (0, 0)),
            pl.BlockSpec(memory_space=pltpu.SMEM),
        ],
        out_specs=[
            pl.BlockSpec((NP, NVN), lambda: (0, 0)),
            pl.BlockSpec((NG, NVN), lambda: (0, 0)),
        ],
        out_shape=[
            jax.ShapeDtypeStruct((NP, NVN), jnp.float32),
            jax.ShapeDtypeStruct((NG, NVN), jnp.float32),
        ],
    )(aff, bp, sb)

    numer = pl.pallas_call(
        _agg_kernel,
        grid=(nblk,),
        in_specs=[
            pl.BlockSpec((BLK, NVN), lambda i: (i, 0)),
            pl.BlockSpec((BLK, HIDDEN), lambda i: (i, 0)),
            pl.BlockSpec((BLK, 1), lambda i: (i, 0)),
        ],
        out_specs=pl.BlockSpec((NG, NVN, HIDDEN), lambda i: (0, 0, 0)),
        out_shape=jax.ShapeDtypeStruct((NG, NVN, HIDDEN), jnp.float32),
    )(w2, h, bp)

    out = pl.pallas_call(
        _head_kernel,
        in_specs=[
            pl.BlockSpec((NG, NVN, HIDDEN), lambda: (0, 0, 0)),
            pl.BlockSpec((NG, NVN), lambda: (0, 0)),
            pl.BlockSpec((HIDDEN, HIDDEN), lambda: (0, 0)),
            pl.BlockSpec((1, HIDDEN), lambda: (0, 0)),
            pl.BlockSpec((HIDDEN, HIDDEN), lambda: (0, 0)),
            pl.BlockSpec((1, HIDDEN), lambda: (0, 0)),
            pl.BlockSpec((HIDDEN, HIDDEN), lambda: (0, 0)),
            pl.BlockSpec((1, HIDDEN), lambda: (0, 0)),
            pl.BlockSpec((HIDDEN, 128), lambda: (0, 0)),
            pl.BlockSpec((1, 128), lambda: (0, 0)),
        ],
        out_specs=pl.BlockSpec((NG, 128), lambda: (0, 0)),
        out_shape=jax.ShapeDtypeStruct((NG, 128), jnp.float32),
    )(numer, denom, Wv1, bv1[None, :], Wv2, bv2[None, :],
      Wm1, bm1[None, :], Wm2, bm2[None, :])
    return out
